# baseline (speedup 1.0000x reference)
import functools

import jax
import jax.numpy as jnp
from jax import lax
from jax.experimental import pallas as pl
from jax.experimental.pallas import tpu as pltpu
from jax.experimental.pallas import tpu_sc as plsc

N_TOK = 4608
N_EMB = 8192
DIM = 256
TILE_T = 384
GRID = N_TOK // TILE_T
COMMITMENT_COST = 0.25
LOSS_SCALE = (1.0 + COMMITMENT_COST) / (N_TOK * DIM)
INV_N = 1.0 / N_TOK


def _process_stripe(zp, s2, idx_ref, acc_ref, macc_ref, cn_ref):
    zn = jnp.sum(zp * zp, axis=1, keepdims=True)
    cn = cn_ref[...]
    d2 = (zn + cn) - s2
    m = jnp.min(d2, axis=1, keepdims=True)
    col = lax.broadcasted_iota(jnp.int32, (TILE_T, N_EMB), 1).astype(jnp.float32)
    idx = jnp.min(jnp.where(d2 == m, col, jnp.float32(1e30)),
                  axis=1, keepdims=True).astype(jnp.int32)
    idx_ref[...] = idx
    u = m - d2
    p = jnp.exp(u.astype(jnp.bfloat16))
    ones_row = jnp.ones((N_EMB, 1), jnp.bfloat16)
    zsum = lax.dot_general(p, ones_row, (((1,), (0,)), ((), ())),
                           preferred_element_type=jnp.float32)
    r = (1.0 / zsum).astype(jnp.bfloat16)
    colsum = lax.dot_general(r, p, (((0,), (0,)), ((), ())),
                             preferred_element_type=jnp.float32)

    acc_ref[...] += colsum
    macc_ref[...] += m


def _vq_body(z_ref, cb_ref, cn_ref, idx_ref, stats_ref, acc_ref, macc_ref):
    i = pl.program_id(0)

    @pl.when(i == 0)
    def _init():
        acc_ref[...] = jnp.zeros_like(acc_ref)
        macc_ref[...] = jnp.zeros_like(macc_ref)

    z = z_ref[...]
    cb = cb_ref[...]
    s2 = lax.dot_general(z + z, cb, (((1,), (1,)), ((), ())),
                         preferred_element_type=jnp.float32)
    _process_stripe(z, s2, idx_ref, acc_ref, macc_ref, cn_ref)

    @pl.when(i == GRID - 1)
    def _finalize():
        msum = jnp.sum(macc_ref[...], axis=0, keepdims=True)
        cc = LOSS_SCALE * msum
        avg = acc_ref[...] * INV_N
        ent = -jnp.sum(avg * jnp.log(avg + 1e-10),
                       axis=1, keepdims=True)
        se = 0.1 * (-ent)
        vq = cc + se
        lanes = lax.broadcasted_iota(jnp.int32, (1, 128), 1)
        stats_ref[...] = jnp.where(
            lanes == 0, vq,
            jnp.where(lanes == 1, cc,
                      jnp.where(lanes == 2, se, jnp.float32(0.0))))


def _vq_call(z2, cb, cn_row):
    return pl.pallas_call(
        _vq_body,
        grid=(GRID,),
        in_specs=[
            pl.BlockSpec((TILE_T, DIM), lambda i: (i, 0)),
            pl.BlockSpec((N_EMB, DIM), lambda i: (0, 0)),
            pl.BlockSpec((1, N_EMB), lambda i: (0, 0)),
        ],
        out_specs=[
            pl.BlockSpec((TILE_T, 1), lambda i: (i, 0)),
            pl.BlockSpec((1, 128), lambda i: (0, 0)),
        ],
        out_shape=[
            jax.ShapeDtypeStruct((N_TOK, 1), jnp.int32),
            jax.ShapeDtypeStruct((1, 128), jnp.float32),
        ],
        scratch_shapes=[
            pltpu.VMEM((1, N_EMB), jnp.float32),
            pltpu.VMEM((TILE_T, 1), jnp.float32),
        ],
    )(z2, cb, cn_row)


def _make_gather():
    info = plsc.get_sparse_core_info()
    nc, ns = info.num_cores, info.num_subcores
    nw = nc * ns
    b_per_w = N_TOK // nw
    mesh = plsc.VectorSubcoreMesh(core_axis_name="c", subcore_axis_name="s")

    @functools.partial(
        pl.kernel, mesh=mesh,
        out_type=jax.ShapeDtypeStruct((N_TOK, DIM), jnp.float32),
        scratch_types=[
            pltpu.VMEM((b_per_w,), jnp.int32),
            pltpu.VMEM((b_per_w, DIM), jnp.float32),
            pltpu.SemaphoreType.DMA,
        ],
    )
    def gather_rows(table_hbm, idx_hbm, out_hbm, idx_v, rows_v, sem):
        wid = lax.axis_index("s") * nc + lax.axis_index("c")
        base = wid * b_per_w
        pltpu.sync_copy(idx_hbm.at[pl.ds(base, b_per_w)], idx_v)
        pltpu.async_copy(table_hbm.at[idx_v], rows_v, sem).wait()
        pltpu.sync_copy(rows_v, out_hbm.at[pl.ds(base, b_per_w)])

    return gather_rows


def kernel(z, codebook_weight):
    B, L, D = z.shape
    z2 = z.reshape(-1, D)
    cn_row = jnp.sum(codebook_weight ** 2, axis=1).reshape(1, -1)
    idx2, stats = _vq_call(z2, codebook_weight, cn_row)
    idx_flat = idx2.reshape(-1)
    zq_rows = _make_gather()(codebook_weight, idx_flat)
    z_q = zq_rows.reshape(B, L, D)
    indices = idx_flat.reshape(B, L)
    vq_loss = stats[0, 0]
    codebook_commitment_loss = stats[0, 1]
    scaled_entropy_loss = stats[0, 2]
    return (z_q, indices, vq_loss, codebook_commitment_loss, scaled_entropy_loss)


# speedup vs baseline: 1.2155x; 1.2155x over previous
import functools

import jax
import jax.numpy as jnp
from jax import lax
from jax.experimental import pallas as pl
from jax.experimental.pallas import tpu as pltpu
from jax.experimental.pallas import tpu_sc as plsc

N_TOK = 4608
N_EMB = 8192
DIM = 256
TILE_T = 256
GRID = N_TOK // TILE_T
COMMITMENT_COST = 0.25
LOSS_SCALE = (1.0 + COMMITMENT_COST) / (N_TOK * DIM)
INV_N = 1.0 / N_TOK


def _process_stripe(zp, s2, idx_ref, acc_ref, macc_ref, cn_ref):
    zn = jnp.sum(zp * zp, axis=1, keepdims=True)
    cn = cn_ref[...]
    d2 = (zn + cn) - s2
    m = jnp.min(d2, axis=1, keepdims=True)
    col = lax.broadcasted_iota(jnp.int32, (TILE_T, N_EMB), 1).astype(jnp.float32)
    idx = jnp.min(jnp.where(d2 == m, col, jnp.float32(1e30)),
                  axis=1, keepdims=True).astype(jnp.int32)
    idx_ref[...] = idx
    u = m - d2
    p = jnp.exp(u.astype(jnp.bfloat16))
    ones_row = jnp.ones((N_EMB, 1), jnp.bfloat16)
    zsum = lax.dot_general(p, ones_row, (((1,), (0,)), ((), ())),
                           preferred_element_type=jnp.float32)
    r = (1.0 / zsum).astype(jnp.bfloat16)
    colsum = lax.dot_general(r, p, (((0,), (0,)), ((), ())),
                             preferred_element_type=jnp.float32)

    acc_ref[...] += colsum
    macc_ref[...] += m


def _vq_body(z_ref, cb_ref, cn_ref, idx_ref, stats_ref, acc_ref, macc_ref):
    i = pl.program_id(0)

    @pl.when(i == 0)
    def _init():
        acc_ref[...] = jnp.zeros_like(acc_ref)
        macc_ref[...] = jnp.zeros_like(macc_ref)

    z = z_ref[...]
    cb = cb_ref[...]
    s2 = lax.dot_general(z + z, cb, (((1,), (1,)), ((), ())),
                         preferred_element_type=jnp.float32)
    _process_stripe(z, s2, idx_ref, acc_ref, macc_ref, cn_ref)

    @pl.when(i == GRID - 1)
    def _finalize():
        msum = jnp.sum(macc_ref[...], axis=0, keepdims=True)
        cc = LOSS_SCALE * msum
        avg = acc_ref[...] * INV_N
        ent = -jnp.sum(avg * jnp.log(avg + 1e-10),
                       axis=1, keepdims=True)
        se = 0.1 * (-ent)
        vq = cc + se
        lanes = lax.broadcasted_iota(jnp.int32, (1, 128), 1)
        stats_ref[...] = jnp.where(
            lanes == 0, vq,
            jnp.where(lanes == 1, cc,
                      jnp.where(lanes == 2, se, jnp.float32(0.0))))


def _vq_call(z2, cb, cn_row):
    return pl.pallas_call(
        _vq_body,
        grid=(GRID,),
        in_specs=[
            pl.BlockSpec((TILE_T, DIM), lambda i: (i, 0)),
            pl.BlockSpec((N_EMB, DIM), lambda i: (0, 0)),
            pl.BlockSpec((1, N_EMB), lambda i: (0, 0)),
        ],
        out_specs=[
            pl.BlockSpec((TILE_T, 1), lambda i: (i, 0)),
            pl.BlockSpec((1, 128), lambda i: (0, 0)),
        ],
        out_shape=[
            jax.ShapeDtypeStruct((N_TOK, 1), jnp.int32),
            jax.ShapeDtypeStruct((1, 128), jnp.float32),
        ],
        scratch_shapes=[
            pltpu.VMEM((1, N_EMB), jnp.float32),
            pltpu.VMEM((TILE_T, 1), jnp.float32),
        ],
    )(z2, cb, cn_row)


def _make_gather():
    info = plsc.get_sparse_core_info()
    nc, ns = info.num_cores, info.num_subcores
    nw = nc * ns
    b_per_w = N_TOK // nw
    mesh = plsc.VectorSubcoreMesh(core_axis_name="c", subcore_axis_name="s")

    @functools.partial(
        pl.kernel, mesh=mesh,
        out_type=jax.ShapeDtypeStruct((N_TOK, DIM), jnp.float32),
        scratch_types=[
            pltpu.VMEM((b_per_w,), jnp.int32),
            pltpu.VMEM((b_per_w, DIM), jnp.float32),
            pltpu.SemaphoreType.DMA,
        ],
    )
    def gather_rows(table_hbm, idx_hbm, out_hbm, idx_v, rows_v, sem):
        wid = lax.axis_index("s") * nc + lax.axis_index("c")
        base = wid * b_per_w
        pltpu.sync_copy(idx_hbm.at[pl.ds(base, b_per_w)], idx_v)
        pltpu.async_copy(table_hbm.at[idx_v], rows_v, sem).wait()
        pltpu.sync_copy(rows_v, out_hbm.at[pl.ds(base, b_per_w)])

    return gather_rows


def kernel(z, codebook_weight):
    B, L, D = z.shape
    z2 = z.reshape(-1, D)
    cn_row = jnp.sum(codebook_weight ** 2, axis=1).reshape(1, -1)
    idx2, stats = _vq_call(z2, codebook_weight, cn_row)
    idx_flat = idx2.reshape(-1)
    zq_rows = _make_gather()(codebook_weight, idx_flat)
    z_q = zq_rows.reshape(B, L, D)
    indices = idx_flat.reshape(B, L)
    vq_loss = stats[0, 0]
    codebook_commitment_loss = stats[0, 1]
    scaled_entropy_loss = stats[0, 2]
    return (z_q, indices, vq_loss, codebook_commitment_loss, scaled_entropy_loss)
